# baseline (speedup 1.0000x reference)
import functools

import jax
import jax.numpy as jnp
from jax import lax
from jax.experimental import pallas as pl
from jax.experimental.pallas import tpu as pltpu
from jax.experimental.pallas import tpu_sc as plsc

D = 64
L = 16
SCALE = 8.0
NC, NS = 2, 16
NW = NC * NS
NB = 4

VBLK = 2048
NPAIR = 245
PACKED_ROWS = NPAIR * VBLK


def _table_transpose_scale(table_t):
    def body(i_ref, o_ref):
        o_ref[:, 0:D] = i_ref[:, 0:VBLK].T * SCALE
        o_ref[:, D : 2 * D] = i_ref[:, VBLK : 2 * VBLK].T * SCALE

    return pl.pallas_call(
        body,
        grid=(NPAIR,),
        in_specs=[pl.BlockSpec((D, 2 * VBLK), lambda j: (0, j))],
        out_specs=pl.BlockSpec((VBLK, 2 * D), lambda j: (j, 0)),
        out_shape=jax.ShapeDtypeStruct((PACKED_ROWS, 2 * D), jnp.float32),
    )(table_t)


def _emb_call(xp, table_rm, n_st, n_bt):
    mesh = plsc.VectorSubcoreMesh(core_axis_name="c", subcore_axis_name="s")
    n_s = n_st * 8

    @functools.partial(
        pl.kernel,
        mesh=mesh,
        out_type=jax.ShapeDtypeStruct((n_s, D // 8, n_bt, 8, 128), jnp.float32),
        scratch_types=[
            pltpu.VMEM((n_st, 8, 128), jnp.int32),
            pltpu.VMEM((NB, 128, D), jnp.float32),
            pltpu.VMEM((NB, D // 8, 8, 128), jnp.float32),
        ]
        + [pltpu.SemaphoreType.DMA] * (2 * NB),
        compiler_params=pltpu.CompilerParams(
            use_tc_tiling_on_sc=False, needs_layout_passes=False
        ),
    )
    def emb(xp_hbm, table_hbm, out_hbm, idx_v, gbuf, sbuf, *sems):
        gsems, ssems = sems[:NB], sems[NB:]
        w = lax.axis_index("s") * NC + lax.axis_index("c")
        pltpu.sync_copy(xp_hbm.at[:, w], idx_v)

        @plsc.parallel_loop(0, n_st, step=1)
        def _(st):
            for sr in range(8):
                for j in range(128 // L):
                    sl = pl.ds(j * L, L)
                    v = idx_v[st, sr, sl]
                    k = lax.shift_right_logical(v, 12)
                    t = lax.bitwise_and(v, VBLK - 1)
                    h = lax.bitwise_and(lax.shift_right_logical(v, 11), 1)
                    idx_v[st, sr, sl] = (
                        lax.shift_left(k, 12) + lax.shift_left(t, 1) + h
                    )

        iota16 = lax.iota(jnp.int32, 16)
        zeros16 = jnp.full((L,), 0, jnp.int32)
        ones16 = jnp.full((L,), 1, jnp.int32)

        def gather(s, b):
            pltpu.async_copy(
                table_hbm.at[idx_v.at[s // 8, s % 8]], gbuf.at[b], gsems[b]
            )

        def store(s, b, wait):
            cp = pltpu.make_async_copy(
                sbuf.at[b], out_hbm.at[s, :, w], ssems[b]
            )
            cp.wait() if wait else cp.start()

        for b in range(NB):
            gather(b, b)

        def outer(o, carry):
            for b in range(NB):
                s = o * NB + b
                pltpu.make_async_copy(
                    table_hbm.at[idx_v.at[s // 8, s % 8]], gbuf.at[b], gsems[b]
                ).wait()

                @pl.when(o > 0)
                def _():
                    store(s - NB, b, wait=True)

                @plsc.parallel_loop(0, 128 // L, step=1, unroll=2)
                def _(q):
                    rows = iota16 + lax.mul(q, L)
                    off = pl.ds(lax.mul(q, L), L)
                    col = zeros16
                    for d in range(D):
                        vals = plsc.load_gather(gbuf.at[b], [rows, col])
                        sbuf[b, d // 8, d % 8, off] = vals
                        col = col + ones16

                @pl.when(o < n_s // NB - 1)
                def _():
                    gather(s + NB, b)

                store(s, b, wait=False)
            return carry

        lax.fori_loop(0, n_s // NB, outer, 0)

        for b in range(NB):
            store(n_s - NB + b, b, wait=True)

    return emb(xp, table_rm)


def kernel(x, table):
    batch, seq = x.shape
    n_bt = batch // 128
    n_st = seq // 8
    xp = x.T.reshape(n_st, 8, n_bt, 128).transpose(0, 2, 1, 3)
    table_rm = _table_transpose_scale(table.T).reshape(2 * PACKED_ROWS, D)
    out5 = _emb_call(xp, table_rm, n_st, n_bt)
    return out5.transpose(2, 4, 0, 1, 3).reshape(batch, seq, D)


# speedup vs baseline: 1.1704x; 1.1704x over previous
import functools

import jax
import jax.numpy as jnp
from jax import lax
from jax.experimental import pallas as pl
from jax.experimental.pallas import tpu as pltpu
from jax.experimental.pallas import tpu_sc as plsc

D = 64
L = 16
SCALE = 8.0
NC, NS = 2, 16
NW = NC * NS
NB = 4

VBLK = 2048
NPAIR = 245
PACKED_ROWS = NPAIR * VBLK


def _table_transpose_scale(table_t):
    def body(i_ref, o_ref):
        o_ref[:, 0:D] = i_ref[:, 0:VBLK].T * SCALE
        o_ref[:, D : 2 * D] = i_ref[:, VBLK : 2 * VBLK].T * SCALE

    return pl.pallas_call(
        body,
        grid=(NPAIR,),
        in_specs=[pl.BlockSpec((D, 2 * VBLK), lambda j: (0, j))],
        out_specs=pl.BlockSpec((VBLK, 2 * D), lambda j: (j, 0)),
        out_shape=jax.ShapeDtypeStruct((PACKED_ROWS, 2 * D), jnp.float32),
    )(table_t)


def _emb_call(xp, table_rm, n_st, n_bt):
    mesh = plsc.VectorSubcoreMesh(core_axis_name="c", subcore_axis_name="s")
    n_s = n_st * 8

    @functools.partial(
        pl.kernel,
        mesh=mesh,
        out_type=jax.ShapeDtypeStruct((n_s, D // 8, n_bt, 8, 128), jnp.float32),
        scratch_types=[
            pltpu.VMEM((n_st, 8, 128), jnp.int32),
            pltpu.VMEM((NB, 128, D), jnp.float32),
            pltpu.VMEM((NB, D // 8, 8, 128), jnp.float32),
        ]
        + [pltpu.SemaphoreType.DMA] * (2 * NB),
        compiler_params=pltpu.CompilerParams(
            use_tc_tiling_on_sc=False,
            needs_layout_passes=False,
            disable_bounds_checks=True,
        ),
    )
    def emb(xp_hbm, table_hbm, out_hbm, idx_v, gbuf, sbuf, *sems):
        gsems, ssems = sems[:NB], sems[NB:]
        w = lax.axis_index("s") * NC + lax.axis_index("c")
        pltpu.sync_copy(xp_hbm.at[:, w], idx_v)

        @plsc.parallel_loop(0, n_st, step=1)
        def _(st):
            for sr in range(8):
                for j in range(128 // L):
                    sl = pl.ds(j * L, L)
                    v = idx_v[st, sr, sl]
                    k = lax.shift_right_logical(v, 12)
                    t = lax.bitwise_and(v, VBLK - 1)
                    h = lax.bitwise_and(lax.shift_right_logical(v, 11), 1)
                    idx_v[st, sr, sl] = (
                        lax.shift_left(k, 12) + lax.shift_left(t, 1) + h
                    )

        iota16 = lax.iota(jnp.int32, 16)
        zeros16 = jnp.full((L,), 0, jnp.int32)
        rows_q = [iota16 + (q * L) for q in range(128 // L)]

        def gather(s, b):
            pltpu.async_copy(
                table_hbm.at[idx_v.at[s // 8, s % 8]], gbuf.at[b], gsems[b]
            )

        def store(s, b, wait):
            cp = pltpu.make_async_copy(
                sbuf.at[b], out_hbm.at[s, :, w], ssems[b]
            )
            cp.wait() if wait else cp.start()

        for b in range(NB):
            gather(b, b)

        def outer(o, carry):
            for b in range(NB):
                s = o * NB + b
                pltpu.make_async_copy(
                    table_hbm.at[idx_v.at[s // 8, s % 8]], gbuf.at[b], gsems[b]
                ).wait()

                @pl.when(o > 0)
                def _():
                    store(s - NB, b, wait=True)

                @plsc.parallel_loop(0, D, step=1, unroll=2)
                def _(d):
                    dt = lax.shift_right_logical(d, 3)
                    dr = lax.bitwise_and(d, 7)
                    col = zeros16 + d
                    for q in range(128 // L):
                        vals = plsc.load_gather(gbuf.at[b], [rows_q[q], col])
                        sbuf[b, dt, dr, pl.ds(q * L, L)] = vals

                @pl.when(o < n_s // NB - 1)
                def _():
                    gather(s + NB, b)

                store(s, b, wait=False)
            return carry

        lax.fori_loop(0, n_s // NB, outer, 0)

        for b in range(NB):
            store(n_s - NB + b, b, wait=True)

    return emb(xp, table_rm)


def kernel(x, table):
    batch, seq = x.shape
    n_bt = batch // 128
    n_st = seq // 8
    xp = x.T.reshape(n_st, 8, n_bt, 128).transpose(0, 2, 1, 3)
    table_rm = _table_transpose_scale(table.T).reshape(2 * PACKED_ROWS, D)
    out5 = _emb_call(xp, table_rm, n_st, n_bt)
    return out5.transpose(2, 4, 0, 1, 3).reshape(batch, seq, D)


# speedup vs baseline: 1.1955x; 1.0215x over previous
import functools

import jax
import jax.numpy as jnp
from jax import lax
from jax.experimental import pallas as pl
from jax.experimental.pallas import tpu as pltpu
from jax.experimental.pallas import tpu_sc as plsc

D = 64
L = 16
SCALE = 8.0
NC, NS = 2, 16
NW = NC * NS
CHUNK = 128
NBUF = 8
LOOKAHEAD = 4

VBLK = 2048
NPAIR = 245
PACKED_ROWS = NPAIR * VBLK


def _table_transpose_scale(table_t):
    def body(i_ref, o_ref):
        o_ref[:, 0:D] = i_ref[:, 0:VBLK].T * SCALE
        o_ref[:, D : 2 * D] = i_ref[:, VBLK : 2 * VBLK].T * SCALE

    return pl.pallas_call(
        body,
        grid=(NPAIR,),
        in_specs=[pl.BlockSpec((D, 2 * VBLK), lambda j: (0, j))],
        out_specs=pl.BlockSpec((VBLK, 2 * D), lambda j: (j, 0)),
        out_shape=jax.ShapeDtypeStruct((PACKED_ROWS, 2 * D), jnp.float32),
    )(table_t)


def _emb_call(idx, table_rm, tot, n_chunks):
    mesh = plsc.VectorSubcoreMesh(core_axis_name="c", subcore_axis_name="s")

    @functools.partial(
        pl.kernel,
        mesh=mesh,
        out_type=jax.ShapeDtypeStruct((tot, D), jnp.float32),
        scratch_types=[
            pltpu.VMEM((n_chunks, CHUNK), jnp.int32),
            pltpu.VMEM((NBUF, CHUNK, D), jnp.float32),
        ]
        + [pltpu.SemaphoreType.DMA] * (2 * NBUF),
        compiler_params=pltpu.CompilerParams(use_tc_tiling_on_sc=False),
    )
    def emb(idx_hbm, table_hbm, out_hbm, idx_v, gbuf, *sems):
        gsems, ssems = sems[:NBUF], sems[NBUF:]
        wid = lax.axis_index("s") * NC + lax.axis_index("c")
        base = wid * n_chunks
        pltpu.sync_copy(idx_hbm.at[pl.ds(base, n_chunks)], idx_v)

        @plsc.parallel_loop(0, n_chunks, step=1, unroll=2)
        def _(r):
            for j in range(CHUNK // L):
                sl = pl.ds(j * L, L)
                v = idx_v[r, sl]
                k = jax.lax.shift_right_logical(v, 12)
                t = jax.lax.bitwise_and(v, VBLK - 1)
                h = jax.lax.bitwise_and(jax.lax.shift_right_logical(v, 11), 1)
                idx_v[r, sl] = (
                    jax.lax.shift_left(k, 12) + jax.lax.shift_left(t, 1) + h
                )

        def gather(g, b):
            pltpu.async_copy(table_hbm.at[idx_v.at[g]], gbuf.at[b], gsems[b])

        def store(g, b, wait):
            cp = pltpu.make_async_copy(
                gbuf.at[b], out_hbm.at[pl.ds((base + g) * CHUNK, CHUNK)], ssems[b]
            )
            cp.wait() if wait else cp.start()

        for g in range(LOOKAHEAD):
            gather(g, g)

        n_outer = n_chunks // NBUF

        def outer(o, carry):
            for b in range(NBUF):
                g = o * NBUF + b
                pltpu.make_async_copy(
                    table_hbm.at[idx_v.at[g]], gbuf.at[b], gsems[b]
                ).wait()
                store(g, b, wait=False)
                h = g + LOOKAHEAD
                bh = (b + LOOKAHEAD) % NBUF

                def _wait_prev_store():
                    store(g - LOOKAHEAD, bh, wait=True)

                if b >= LOOKAHEAD:
                    _wait_prev_store()
                else:
                    pl.when(o > 0)(_wait_prev_store)

                @pl.when(h < n_chunks)
                def _():
                    gather(h, bh)

            return carry

        lax.fori_loop(0, n_outer, outer, 0)

        for k in range(LOOKAHEAD):
            g = n_chunks - LOOKAHEAD + k
            store(g, g % NBUF, wait=True)

    return emb(idx, table_rm)


def kernel(x, table):
    batch, seq = x.shape
    tot = batch * seq
    n_chunks = tot // (NW * CHUNK)
    idx = x.reshape(NW * n_chunks, CHUNK)
    table_rm = _table_transpose_scale(table.T).reshape(2 * PACKED_ROWS, D)
    out = _emb_call(idx, table_rm, tot, n_chunks)
    return out.reshape(batch, seq, D)
